# Initial kernel scaffold; baseline (speedup 1.0000x reference)
import jax
import jax.numpy as jnp
from jax.experimental import pallas as pl


def kernel(span_features, mention_ids, antecedent_ids, distances, speakers, dist_table, spk_table, W1, b1, W2, b2, W3, b3, epsilon):
    raise NotImplementedError("write your pallas kernel here")


# speedup vs baseline: 5.8380x; 5.8380x over previous
import functools

import jax
import jax.numpy as jnp
from jax import lax
from jax.experimental import pallas as pl
from jax.experimental.pallas import tpu as pltpu
from jax.experimental.pallas import tpu_sc as plsc

N_SPANS = 8192
D = 128
K = 32
P = N_SPANS * K
HID = 150
HIDP = 256
BINS_VALS = (1, 2, 3, 4, 8, 16, 32, 64)

NC, NS = 2, 16
NW = NC * NS
ROWS_TOTAL = 2 * P
ROWS_PER_W = ROWS_TOTAL // NW
CHUNK = 128
NCHUNKS = ROWS_PER_W // CHUNK
KD = 4
NG = NCHUNKS // KD

T = 1024
NT = P // T


def _sc_gather_body(table_hbm, ids_hbm, out_hbm, idx_v, rows_v, gsem, wsem):
    wid = lax.axis_index("s") * NC + lax.axis_index("c")
    base = wid * ROWS_PER_W
    pltpu.sync_copy(ids_hbm.at[wid], idx_v)

    def group(g, _):
        descs = []
        for b in range(KD):
            d = pltpu.async_copy(
                table_hbm.at[idx_v.at[g * KD + b]], rows_v.at[b], gsem)
            descs.append(d)
        wdescs = []
        for b in range(KD):
            descs[b].wait()
            wd = pltpu.async_copy(
                rows_v.at[b],
                out_hbm.at[pl.ds(base + (g * KD + b) * CHUNK, CHUNK)],
                wsem)
            wdescs.append(wd)
        for b in range(KD):
            wdescs[b].wait()
        return 0

    lax.fori_loop(0, NG, group, 0)


@functools.cache
def _build_sc_gather():
    return functools.partial(
        pl.kernel,
        out_type=jax.ShapeDtypeStruct((ROWS_TOTAL, D), jnp.float32),
        mesh=plsc.VectorSubcoreMesh(
            core_axis_name="c", subcore_axis_name="s",
            num_cores=NC, num_subcores=NS),
        scratch_types=[
            pltpu.VMEM((NCHUNKS, CHUNK), jnp.int32),
            pltpu.VMEM((KD, CHUNK, D), jnp.float32),
            pltpu.SemaphoreType.DMA,
            pltpu.SemaphoreType.DMA,
        ],
    )(_sc_gather_body)


def _sc_gather(span_features, ids3):
    return _build_sc_gather()(span_features, ids3)


def _tc_mlp_body(gi_ref, gj_ref, dist_ref, spk_ref, bins_ref,
                 w1a_ref, w1b_ref, w1c_ref, otw_ref, w2_ref, b2_ref,
                 w3_ref, b3_ref, out_ref):
    gi = gi_ref[...]
    gj = gj_ref[...]
    col = lax.broadcasted_iota(jnp.int32, (T, 16), 1)
    distb = jnp.broadcast_to(dist_ref[...], (T, 16))
    spkb = jnp.broadcast_to(spk_ref[...], (T, 16))
    binsb = jnp.broadcast_to(bins_ref[...], (T, 16))
    m = jnp.where(
        col < 8, (distb > binsb).astype(jnp.float32),
        jnp.where(col < 11, (spkb == (col - 8)).astype(jnp.float32),
                  jnp.where(col == 11, 1.0, 0.0)))
    acc = jnp.dot(gi, w1a_ref[...], preferred_element_type=jnp.float32)
    acc += jnp.dot(gj, w1b_ref[...], preferred_element_type=jnp.float32)
    acc += jnp.dot(gi * gj, w1c_ref[...], preferred_element_type=jnp.float32)
    acc += jnp.dot(m, otw_ref[...], preferred_element_type=jnp.float32)
    h = jnp.maximum(acc, 0.0)
    h2 = jnp.dot(h, w2_ref[...], preferred_element_type=jnp.float32)
    h2 = jnp.maximum(h2 + b2_ref[...], 0.0)
    r = jnp.dot(h2, w3_ref[...], preferred_element_type=jnp.float32)
    r = r + b3_ref[...]
    out_ref[...] = r[:, :2]


def _tc_mlp(gathered, dist2, spk2, binspad, w1a, w1b, w1c, otw, w2p, b2p,
            w3p, b3p):
    full = lambda shape: pl.BlockSpec(shape, lambda i: (0, 0))
    return pl.pallas_call(
        _tc_mlp_body,
        grid=(NT,),
        in_specs=[
            pl.BlockSpec((T, D), lambda i: (i, 0)),
            pl.BlockSpec((T, D), lambda i: (i + NT, 0)),
            pl.BlockSpec((T, 1), lambda i: (i, 0)),
            pl.BlockSpec((T, 1), lambda i: (i, 0)),
            full((1, 16)),
            full((D, HIDP)), full((D, HIDP)), full((D, HIDP)),
            full((16, HIDP)),
            full((HIDP, HIDP)), full((1, HIDP)),
            full((HIDP, 128)), full((1, 128)),
        ],
        out_specs=pl.BlockSpec((T, 2), lambda i: (i, 0)),
        out_shape=jax.ShapeDtypeStruct((P, 2), jnp.float32),
        compiler_params=pltpu.CompilerParams(
            dimension_semantics=("arbitrary",)),
    )(gathered, gathered, dist2, spk2, binspad, w1a, w1b, w1c, otw, w2p,
      b2p, w3p, b3p)


def kernel(span_features, mention_ids, antecedent_ids, distances, speakers,
           dist_table, spk_table, W1, b1, W2, b2, W3, b3, epsilon):
    f32 = jnp.float32
    w1a = jnp.zeros((D, HIDP), f32).at[:, :HID].set(W1[0:D])
    w1b = jnp.zeros((D, HIDP), f32).at[:, :HID].set(W1[D:2 * D])
    w1c = jnp.zeros((D, HIDP), f32).at[:, :HID].set(W1[2 * D:3 * D])
    w1d = W1[3 * D:3 * D + 20]
    w1s = W1[3 * D + 20:3 * D + 40]
    deltas = (dist_table[1:9] - dist_table[0:8]) @ w1d
    spkrows = spk_table @ w1s
    const_row = dist_table[0] @ w1d + b1
    otw = jnp.zeros((16, HIDP), f32)
    otw = otw.at[0:8, :HID].set(deltas)
    otw = otw.at[8:11, :HID].set(spkrows)
    otw = otw.at[11, :HID].set(const_row)
    w2p = jnp.zeros((HIDP, HIDP), f32).at[:HID, :HID].set(W2)
    b2p = jnp.zeros((1, HIDP), f32).at[0, :HID].set(b2)
    w3p = jnp.zeros((HIDP, 128), f32).at[:HID, :2].set(W3)
    b3p = jnp.zeros((1, 128), f32).at[0, :2].set(b3)
    binspad = jnp.full((1, 16), 2**30, jnp.int32).at[0, :8].set(
        jnp.array(BINS_VALS, jnp.int32))

    ids = jnp.concatenate([mention_ids.astype(jnp.int32),
                           antecedent_ids.astype(jnp.int32)])
    ids3 = ids.reshape(NW, NCHUNKS, CHUNK)
    gathered = _sc_gather(span_features, ids3)

    dist2 = distances.astype(jnp.int32).reshape(P, 1)
    spk2 = speakers.astype(jnp.int32).reshape(P, 1)
    scores = _tc_mlp(gathered, dist2, spk2, binspad, w1a, w1b, w1c, otw,
                     w2p, b2p, w3p, b3p)

    scores = scores.reshape(N_SPANS, K, 2)
    eps = jnp.broadcast_to(epsilon.reshape(1, 1, 2), (N_SPANS, 1, 2))
    return jnp.concatenate([scores, eps], axis=1)
